# initial kernel scaffold (unmeasured)
import jax
import jax.numpy as jnp
from jax import lax
from jax.experimental import pallas as pl
from jax.experimental.pallas import tpu as pltpu

N_DEV = 4


def kernel(O, Wo):
    b, s, h, d = O.shape
    k = h * d
    n = Wo.shape[1]
    s_chunk = s // N_DEV
    o = O.reshape(b, s, k)

    def body(o_ref, wo_ref, out_ref, comm_ref, pacc_ref, send_sems, recv_sems):
        my = lax.axis_index("i")
        left = lax.rem(my + N_DEV - 1, N_DEV)
        right = lax.rem(my + 1, N_DEV)

        barrier_sem = pltpu.get_barrier_semaphore()
        for nbr in (left, right):
            pl.semaphore_signal(
                barrier_sem, inc=1,
                device_id=(nbr,), device_id_type=pl.DeviceIdType.MESH,
            )
        pl.semaphore_wait(barrier_sem, 2)

        def partial_into(dst, c):
            row0 = c * s_chunk
            for bb in range(b):
                dst[bb] = jnp.dot(
                    o_ref[bb, pl.ds(row0, s_chunk), :],
                    wo_ref[...],
                    preferred_element_type=jnp.float32,
                )

        partial_into(comm_ref.at[0], left)

        for t in range(N_DEV - 1):
            rdma = pltpu.make_async_remote_copy(
                src_ref=comm_ref.at[t],
                dst_ref=comm_ref.at[t + 1],
                send_sem=send_sems.at[t],
                recv_sem=recv_sems.at[t],
                device_id=(right,),
                device_id_type=pl.DeviceIdType.MESH,
            )
            rdma.start()
            c_in = lax.rem(my + (N_DEV + 2 - t), N_DEV)
            partial_into(pacc_ref, c_in)
            rdma.wait()
            if t < N_DEV - 2:
                comm_ref[t + 1] = comm_ref[t + 1] + pacc_ref[...]
            else:
                out_ref[...] = comm_ref[t + 1] + pacc_ref[...]

    out_shape = jax.ShapeDtypeStruct((b, s_chunk, n), jnp.float32)
    return pl.pallas_call(
        body,
        out_shape=out_shape,
        in_specs=[
            pl.BlockSpec(memory_space=pltpu.VMEM),
            pl.BlockSpec(memory_space=pltpu.VMEM),
        ],
        out_specs=pl.BlockSpec(memory_space=pltpu.VMEM),
        scratch_shapes=[
            pltpu.VMEM((N_DEV, b, s_chunk, n), jnp.float32),
            pltpu.VMEM((b, s_chunk, n), jnp.float32),
            pltpu.SemaphoreType.DMA((N_DEV - 1,)),
            pltpu.SemaphoreType.DMA((N_DEV - 1,)),
        ],
        compiler_params=pltpu.CompilerParams(collective_id=0),
    )(o, Wo)


# baseline (device time: 306298 ns/iter reference)
import jax
import jax.numpy as jnp
from jax import lax
from jax.experimental import pallas as pl
from jax.experimental.pallas import tpu as pltpu

N_DEV = 4


def kernel(O, Wo):
    b, s, h, d = O.shape
    k = h * d
    n = Wo.shape[1]
    s_chunk = s // N_DEV
    o = O.reshape(b, s, k)

    def body(o_ref, wo_ref, out_ref, comm_ref, pacc_ref, send_sems, recv_sems):
        my = lax.axis_index("i")
        left = lax.rem(my + N_DEV - 1, N_DEV)
        right = lax.rem(my + 1, N_DEV)

        barrier_sem = pltpu.get_barrier_semaphore()
        for nbr in (left, right):
            pl.semaphore_signal(
                barrier_sem, inc=1,
                device_id=(nbr,), device_id_type=pl.DeviceIdType.MESH,
            )
        pl.semaphore_wait(barrier_sem, 2)

        def partial_into(dst, c):
            row0 = c * s_chunk
            for bb in range(b):
                dst[bb] = jnp.dot(
                    o_ref[bb, pl.ds(row0, s_chunk), :],
                    wo_ref[...],
                    preferred_element_type=jnp.float32,
                )

        partial_into(comm_ref.at[0], left)

        for t in range(N_DEV - 1):
            rdma = pltpu.make_async_remote_copy(
                src_ref=comm_ref.at[t],
                dst_ref=comm_ref.at[t + 1],
                send_sem=send_sems.at[t],
                recv_sem=recv_sems.at[t],
                device_id=(right,),
                device_id_type=pl.DeviceIdType.MESH,
            )
            rdma.start()
            c_in = lax.rem(my + (N_DEV + 2 - t), N_DEV)
            partial_into(pacc_ref, c_in)
            rdma.wait()
            if t < N_DEV - 2:
                comm_ref[t + 1] = comm_ref[t + 1] + pacc_ref[...]
            else:
                out_ref[...] = comm_ref[t + 1] + pacc_ref[...]

    out_shape = jax.ShapeDtypeStruct((b, s_chunk, n), jnp.float32)
    return pl.pallas_call(
        body,
        out_shape=out_shape,
        in_specs=[
            pl.BlockSpec(memory_space=pltpu.VMEM),
            pl.BlockSpec(memory_space=pltpu.VMEM),
        ],
        out_specs=pl.BlockSpec(memory_space=pltpu.VMEM),
        scratch_shapes=[
            pltpu.VMEM((N_DEV, b, s_chunk, n), jnp.float32),
            pltpu.VMEM((b, s_chunk, n), jnp.float32),
            pltpu.SemaphoreType.DMA((N_DEV - 1,)),
            pltpu.SemaphoreType.DMA((N_DEV - 1,)),
        ],
        compiler_params=pltpu.CompilerParams(
            collective_id=0,
            vmem_limit_bytes=100 * 1024 * 1024,
        ),
    )(o, Wo)


# device time: 171610 ns/iter; 1.7848x vs baseline; 1.7848x over previous
import jax
import jax.numpy as jnp
from jax import lax
from jax.experimental import pallas as pl
from jax.experimental.pallas import tpu as pltpu

N_DEV = 4


def kernel(O, Wo):
    b, s, h, d = O.shape
    k = h * d
    n = Wo.shape[1]
    nh = n // 2
    s_chunk = s // N_DEV
    o = O.reshape(b, s, k)

    def body(o_ref, wo_ref, out_ref, cw_ref, ccw_ref, pa_ref, pb_ref,
             cw_send_sems, cw_recv_sems, ccw_send_sems, ccw_recv_sems):
        my = lax.axis_index("i")
        left = lax.rem(my + N_DEV - 1, N_DEV)
        right = lax.rem(my + 1, N_DEV)

        barrier_sem = pltpu.get_barrier_semaphore()
        for nbr in (left, right):
            pl.semaphore_signal(
                barrier_sem, inc=1,
                device_id=(nbr,), device_id_type=pl.DeviceIdType.MESH,
            )
        pl.semaphore_wait(barrier_sem, 2)

        def partial_a(dst, c):
            row0 = c * s_chunk
            for bb in range(b):
                dst[bb] = jnp.dot(
                    o_ref[bb, pl.ds(row0, s_chunk), :],
                    wo_ref[:, 0:nh],
                    preferred_element_type=jnp.float32,
                )

        def partial_b(dst, c):
            row0 = c * s_chunk
            for bb in range(b):
                dst[bb] = jnp.dot(
                    o_ref[bb, pl.ds(row0, s_chunk), :],
                    wo_ref[:, nh:n],
                    preferred_element_type=jnp.float32,
                )

        partial_a(cw_ref.at[0], left)
        partial_b(ccw_ref.at[0], right)

        for t in range(N_DEV - 1):
            cw_rdma = pltpu.make_async_remote_copy(
                src_ref=cw_ref.at[t],
                dst_ref=cw_ref.at[t + 1],
                send_sem=cw_send_sems.at[t],
                recv_sem=cw_recv_sems.at[t],
                device_id=(right,),
                device_id_type=pl.DeviceIdType.MESH,
            )
            ccw_rdma = pltpu.make_async_remote_copy(
                src_ref=ccw_ref.at[t],
                dst_ref=ccw_ref.at[t + 1],
                send_sem=ccw_send_sems.at[t],
                recv_sem=ccw_recv_sems.at[t],
                device_id=(left,),
                device_id_type=pl.DeviceIdType.MESH,
            )
            cw_rdma.start()
            ccw_rdma.start()
            c_cw = lax.rem(my + (N_DEV + 2 - t), N_DEV)
            c_ccw = lax.rem(my + 2 + t, N_DEV)
            partial_a(pa_ref, c_cw)
            partial_b(pb_ref, c_ccw)
            cw_rdma.wait()
            ccw_rdma.wait()
            if t < N_DEV - 2:
                cw_ref[t + 1] = cw_ref[t + 1] + pa_ref[...]
                ccw_ref[t + 1] = ccw_ref[t + 1] + pb_ref[...]
            else:
                out_ref[:, :, 0:nh] = cw_ref[t + 1] + pa_ref[...]
                out_ref[:, :, nh:n] = ccw_ref[t + 1] + pb_ref[...]

    out_shape = jax.ShapeDtypeStruct((b, s_chunk, n), jnp.float32)
    return pl.pallas_call(
        body,
        out_shape=out_shape,
        in_specs=[
            pl.BlockSpec(memory_space=pltpu.VMEM),
            pl.BlockSpec(memory_space=pltpu.VMEM),
        ],
        out_specs=pl.BlockSpec(memory_space=pltpu.VMEM),
        scratch_shapes=[
            pltpu.VMEM((N_DEV, b, s_chunk, nh), jnp.float32),
            pltpu.VMEM((N_DEV, b, s_chunk, nh), jnp.float32),
            pltpu.VMEM((b, s_chunk, nh), jnp.float32),
            pltpu.VMEM((b, s_chunk, nh), jnp.float32),
            pltpu.SemaphoreType.DMA((N_DEV - 1,)),
            pltpu.SemaphoreType.DMA((N_DEV - 1,)),
            pltpu.SemaphoreType.DMA((N_DEV - 1,)),
            pltpu.SemaphoreType.DMA((N_DEV - 1,)),
        ],
        compiler_params=pltpu.CompilerParams(
            collective_id=0,
            vmem_limit_bytes=100 * 1024 * 1024,
        ),
    )(o, Wo)


# device time: 165025 ns/iter; 1.8561x vs baseline; 1.0399x over previous
import jax
import jax.numpy as jnp
from jax import lax
from jax.experimental import pallas as pl
from jax.experimental.pallas import tpu as pltpu

N_DEV = 4
N_SUB = 2


def kernel(O, Wo):
    b, s, h, d = O.shape
    k = h * d
    n = Wo.shape[1]
    nh = n // 2
    s_chunk = s // N_DEV
    bsub = b // N_SUB
    o = O.reshape(b, s, k)

    def body(o_ref, wo_ref, out_ref, cw_ref, ccw_ref, pa_ref, pb_ref,
             cw_ssems, cw_rsems, ccw_ssems, ccw_rsems):
        my = lax.axis_index("i")
        left = lax.rem(my + N_DEV - 1, N_DEV)
        right = lax.rem(my + 1, N_DEV)

        barrier_sem = pltpu.get_barrier_semaphore()
        for nbr in (left, right):
            pl.semaphore_signal(
                barrier_sem, inc=1,
                device_id=(nbr,), device_id_type=pl.DeviceIdType.MESH,
            )
        pl.semaphore_wait(barrier_sem, 2)

        def pslab(dst, c, col0, b0, nb):
            row0 = c * s_chunk
            for bb in range(nb):
                dst[bb] = jnp.dot(
                    o_ref[b0 + bb, pl.ds(row0, s_chunk), :],
                    wo_ref[:, col0:col0 + nh],
                    preferred_element_type=jnp.float32,
                )

        def mk(dir_ref, ssems, rsems, t, j, dev):
            sl = pl.ds(j * bsub, bsub)
            return pltpu.make_async_remote_copy(
                src_ref=dir_ref.at[t, sl],
                dst_ref=dir_ref.at[t + 1, sl],
                send_sem=ssems.at[t, j],
                recv_sem=rsems.at[t, j],
                device_id=(dev,),
                device_id_type=pl.DeviceIdType.MESH,
            )

        for j in range(N_SUB):
            sl = pl.ds(j * bsub, bsub)
            pslab(cw_ref.at[0, sl], left, 0, j * bsub, bsub)
            mk(cw_ref, cw_ssems, cw_rsems, 0, j, right).start()
            pslab(ccw_ref.at[0, sl], right, nh, j * bsub, bsub)
            mk(ccw_ref, ccw_ssems, ccw_rsems, 0, j, left).start()

        for t in range(N_DEV - 1):
            c_cw = lax.rem(my + (N_DEV + 2 - t), N_DEV)
            c_ccw = lax.rem(my + 2 + t, N_DEV)
            pslab(pa_ref, c_cw, 0, 0, b)
            pslab(pb_ref, c_ccw, nh, 0, b)
            for j in range(N_SUB):
                sl = pl.ds(j * bsub, bsub)
                mk(cw_ref, cw_ssems, cw_rsems, t, j, right).wait_recv()
                if t < N_DEV - 2:
                    cw_ref[t + 1, sl] = cw_ref[t + 1, sl] + pa_ref[sl]
                    mk(cw_ref, cw_ssems, cw_rsems, t + 1, j, right).start()
                else:
                    out_ref[sl, :, 0:nh] = cw_ref[t + 1, sl] + pa_ref[sl]
                mk(ccw_ref, ccw_ssems, ccw_rsems, t, j, left).wait_recv()
                if t < N_DEV - 2:
                    ccw_ref[t + 1, sl] = ccw_ref[t + 1, sl] + pb_ref[sl]
                    mk(ccw_ref, ccw_ssems, ccw_rsems, t + 1, j, left).start()
                else:
                    out_ref[sl, :, nh:n] = ccw_ref[t + 1, sl] + pb_ref[sl]

        for t in range(N_DEV - 1):
            for j in range(N_SUB):
                mk(cw_ref, cw_ssems, cw_rsems, t, j, right).wait_send()
                mk(ccw_ref, ccw_ssems, ccw_rsems, t, j, left).wait_send()

    out_shape = jax.ShapeDtypeStruct((b, s_chunk, n), jnp.float32)
    return pl.pallas_call(
        body,
        out_shape=out_shape,
        in_specs=[
            pl.BlockSpec(memory_space=pltpu.VMEM),
            pl.BlockSpec(memory_space=pltpu.VMEM),
        ],
        out_specs=pl.BlockSpec(memory_space=pltpu.VMEM),
        scratch_shapes=[
            pltpu.VMEM((N_DEV, b, s_chunk, nh), jnp.float32),
            pltpu.VMEM((N_DEV, b, s_chunk, nh), jnp.float32),
            pltpu.VMEM((b, s_chunk, nh), jnp.float32),
            pltpu.VMEM((b, s_chunk, nh), jnp.float32),
            pltpu.SemaphoreType.DMA((N_DEV - 1, N_SUB)),
            pltpu.SemaphoreType.DMA((N_DEV - 1, N_SUB)),
            pltpu.SemaphoreType.DMA((N_DEV - 1, N_SUB)),
            pltpu.SemaphoreType.DMA((N_DEV - 1, N_SUB)),
        ],
        compiler_params=pltpu.CompilerParams(
            collective_id=0,
            vmem_limit_bytes=100 * 1024 * 1024,
        ),
    )(o, Wo)


# device time: 99033 ns/iter; 3.0929x vs baseline; 1.6664x over previous
import jax
import jax.numpy as jnp
from jax import lax
from jax.experimental import pallas as pl
from jax.experimental.pallas import tpu as pltpu

N_DEV = 4
N_SUB = 2


def kernel(O, Wo):
    b, s, h, d = O.shape
    k = h * d
    n = Wo.shape[1]
    nh = n // 2
    s_chunk = s // N_DEV
    bsub = b // N_SUB
    o = O.reshape(b, s, k).astype(jnp.bfloat16)
    wo = Wo.astype(jnp.bfloat16)

    def body(o_ref, wo_ref, out_ref, cw_ref, ccw_ref, pa_ref, pb_ref,
             cw_ssems, cw_rsems, ccw_ssems, ccw_rsems):
        my = lax.axis_index("i")
        left = lax.rem(my + N_DEV - 1, N_DEV)
        right = lax.rem(my + 1, N_DEV)

        barrier_sem = pltpu.get_barrier_semaphore()
        for nbr in (left, right):
            pl.semaphore_signal(
                barrier_sem, inc=1,
                device_id=(nbr,), device_id_type=pl.DeviceIdType.MESH,
            )
        pl.semaphore_wait(barrier_sem, 2)

        def pslab(dst, c, col0, b0, nb, dtype):
            row0 = c * s_chunk
            for bb in range(nb):
                dst[bb] = jnp.dot(
                    o_ref[b0 + bb, pl.ds(row0, s_chunk), :],
                    wo_ref[:, col0:col0 + nh],
                    preferred_element_type=jnp.float32,
                ).astype(dtype)

        def mk(dir_ref, ssems, rsems, t, j, dev):
            sl = pl.ds(j * bsub, bsub)
            return pltpu.make_async_remote_copy(
                src_ref=dir_ref.at[t, sl],
                dst_ref=dir_ref.at[t + 1, sl],
                send_sem=ssems.at[t, j],
                recv_sem=rsems.at[t, j],
                device_id=(dev,),
                device_id_type=pl.DeviceIdType.MESH,
            )

        for j in range(N_SUB):
            sl = pl.ds(j * bsub, bsub)
            pslab(cw_ref.at[0, sl], left, 0, j * bsub, bsub, jnp.bfloat16)
            mk(cw_ref, cw_ssems, cw_rsems, 0, j, right).start()
            pslab(ccw_ref.at[0, sl], right, nh, j * bsub, bsub, jnp.bfloat16)
            mk(ccw_ref, ccw_ssems, ccw_rsems, 0, j, left).start()

        for t in range(N_DEV - 1):
            c_cw = lax.rem(my + (N_DEV + 2 - t), N_DEV)
            c_ccw = lax.rem(my + 2 + t, N_DEV)
            pslab(pa_ref, c_cw, 0, 0, b, jnp.float32)
            pslab(pb_ref, c_ccw, nh, 0, b, jnp.float32)
            for j in range(N_SUB):
                sl = pl.ds(j * bsub, bsub)
                mk(cw_ref, cw_ssems, cw_rsems, t, j, right).wait_recv()
                if t < N_DEV - 2:
                    cw_ref[t + 1, sl] = (
                        cw_ref[t + 1, sl] + pa_ref[sl]
                    ).astype(jnp.bfloat16)
                    mk(cw_ref, cw_ssems, cw_rsems, t + 1, j, right).start()
                else:
                    out_ref[sl, :, 0:nh] = cw_ref[t + 1, sl] + pa_ref[sl]
                mk(ccw_ref, ccw_ssems, ccw_rsems, t, j, left).wait_recv()
                if t < N_DEV - 2:
                    ccw_ref[t + 1, sl] = (
                        ccw_ref[t + 1, sl] + pb_ref[sl]
                    ).astype(jnp.bfloat16)
                    mk(ccw_ref, ccw_ssems, ccw_rsems, t + 1, j, left).start()
                else:
                    out_ref[sl, :, nh:n] = ccw_ref[t + 1, sl] + pb_ref[sl]

        for t in range(N_DEV - 1):
            for j in range(N_SUB):
                mk(cw_ref, cw_ssems, cw_rsems, t, j, right).wait_send()
                mk(ccw_ref, ccw_ssems, ccw_rsems, t, j, left).wait_send()

    out_shape = jax.ShapeDtypeStruct((b, s_chunk, n), jnp.float32)
    return pl.pallas_call(
        body,
        out_shape=out_shape,
        in_specs=[
            pl.BlockSpec(memory_space=pltpu.VMEM),
            pl.BlockSpec(memory_space=pltpu.VMEM),
        ],
        out_specs=pl.BlockSpec(memory_space=pltpu.VMEM),
        scratch_shapes=[
            pltpu.VMEM((N_DEV, b, s_chunk, nh), jnp.bfloat16),
            pltpu.VMEM((N_DEV, b, s_chunk, nh), jnp.bfloat16),
            pltpu.VMEM((b, s_chunk, nh), jnp.float32),
            pltpu.VMEM((b, s_chunk, nh), jnp.float32),
            pltpu.SemaphoreType.DMA((N_DEV - 1, N_SUB)),
            pltpu.SemaphoreType.DMA((N_DEV - 1, N_SUB)),
            pltpu.SemaphoreType.DMA((N_DEV - 1, N_SUB)),
            pltpu.SemaphoreType.DMA((N_DEV - 1, N_SUB)),
        ],
        compiler_params=pltpu.CompilerParams(
            collective_id=0,
            vmem_limit_bytes=100 * 1024 * 1024,
        ),
    )(o, wo)


# device time: 97875 ns/iter; 3.1295x vs baseline; 1.0118x over previous
import jax
import jax.numpy as jnp
from jax import lax
from jax.experimental import pallas as pl
from jax.experimental.pallas import tpu as pltpu

N_DEV = 4
N_SUB = 4


def kernel(O, Wo):
    b, s, h, d = O.shape
    k = h * d
    n = Wo.shape[1]
    nh = n // 2
    s_chunk = s // N_DEV
    bsub = b // N_SUB
    o = O.reshape(b, s, k)

    def body(o_ref, wo_ref, out_ref, ob_ref, wob_ref,
             cw_ref, ccw_ref, pa_ref, pb_ref,
             cw_ssems, cw_rsems, ccw_ssems, ccw_rsems):
        my = lax.axis_index("i")
        left = lax.rem(my + N_DEV - 1, N_DEV)
        right = lax.rem(my + 1, N_DEV)

        for bb in range(b):
            ob_ref[bb] = o_ref[bb].astype(jnp.bfloat16)
        wob_ref[...] = wo_ref[...].astype(jnp.bfloat16)

        barrier_sem = pltpu.get_barrier_semaphore()
        for nbr in (left, right):
            pl.semaphore_signal(
                barrier_sem, inc=1,
                device_id=(nbr,), device_id_type=pl.DeviceIdType.MESH,
            )
        pl.semaphore_wait(barrier_sem, 2)

        def pslab(dst, c, col0, b0, nb, dtype):
            row0 = c * s_chunk
            for bb in range(nb):
                dst[bb] = jnp.dot(
                    ob_ref[b0 + bb, pl.ds(row0, s_chunk), :],
                    wob_ref[:, col0:col0 + nh],
                    preferred_element_type=jnp.float32,
                ).astype(dtype)

        def mk(dir_ref, ssems, rsems, t, j, dev):
            sl = pl.ds(j * bsub, bsub)
            return pltpu.make_async_remote_copy(
                src_ref=dir_ref.at[t, sl],
                dst_ref=dir_ref.at[t + 1, sl],
                send_sem=ssems.at[t, j],
                recv_sem=rsems.at[t, j],
                device_id=(dev,),
                device_id_type=pl.DeviceIdType.MESH,
            )

        for j in range(N_SUB):
            sl = pl.ds(j * bsub, bsub)
            pslab(cw_ref.at[0, sl], left, 0, j * bsub, bsub, jnp.bfloat16)
            mk(cw_ref, cw_ssems, cw_rsems, 0, j, right).start()
            pslab(ccw_ref.at[0, sl], right, nh, j * bsub, bsub, jnp.bfloat16)
            mk(ccw_ref, ccw_ssems, ccw_rsems, 0, j, left).start()

        for t in range(N_DEV - 1):
            c_cw = lax.rem(my + (N_DEV + 2 - t), N_DEV)
            c_ccw = lax.rem(my + 2 + t, N_DEV)
            pslab(pa_ref, c_cw, 0, 0, b, jnp.float32)
            pslab(pb_ref, c_ccw, nh, 0, b, jnp.float32)
            for j in range(N_SUB):
                sl = pl.ds(j * bsub, bsub)
                mk(cw_ref, cw_ssems, cw_rsems, t, j, right).wait_recv()
                if t < N_DEV - 2:
                    cw_ref[t + 1, sl] = (
                        cw_ref[t + 1, sl] + pa_ref[sl]
                    ).astype(jnp.bfloat16)
                    mk(cw_ref, cw_ssems, cw_rsems, t + 1, j, right).start()
                else:
                    out_ref[sl, :, 0:nh] = cw_ref[t + 1, sl] + pa_ref[sl]
                mk(ccw_ref, ccw_ssems, ccw_rsems, t, j, left).wait_recv()
                if t < N_DEV - 2:
                    ccw_ref[t + 1, sl] = (
                        ccw_ref[t + 1, sl] + pb_ref[sl]
                    ).astype(jnp.bfloat16)
                    mk(ccw_ref, ccw_ssems, ccw_rsems, t + 1, j, left).start()
                else:
                    out_ref[sl, :, nh:n] = ccw_ref[t + 1, sl] + pb_ref[sl]

        for t in range(N_DEV - 1):
            for j in range(N_SUB):
                mk(cw_ref, cw_ssems, cw_rsems, t, j, right).wait_send()
                mk(ccw_ref, ccw_ssems, ccw_rsems, t, j, left).wait_send()

    out_shape = jax.ShapeDtypeStruct((b, s_chunk, n), jnp.float32)
    return pl.pallas_call(
        body,
        out_shape=out_shape,
        in_specs=[
            pl.BlockSpec(memory_space=pltpu.VMEM),
            pl.BlockSpec(memory_space=pltpu.VMEM),
        ],
        out_specs=pl.BlockSpec(memory_space=pltpu.VMEM),
        scratch_shapes=[
            pltpu.VMEM((b, s, k), jnp.bfloat16),
            pltpu.VMEM((k, n), jnp.bfloat16),
            pltpu.VMEM((N_DEV, b, s_chunk, nh), jnp.bfloat16),
            pltpu.VMEM((N_DEV, b, s_chunk, nh), jnp.bfloat16),
            pltpu.VMEM((b, s_chunk, nh), jnp.float32),
            pltpu.VMEM((b, s_chunk, nh), jnp.float32),
            pltpu.SemaphoreType.DMA((N_DEV - 1, N_SUB)),
            pltpu.SemaphoreType.DMA((N_DEV - 1, N_SUB)),
            pltpu.SemaphoreType.DMA((N_DEV - 1, N_SUB)),
            pltpu.SemaphoreType.DMA((N_DEV - 1, N_SUB)),
        ],
        compiler_params=pltpu.CompilerParams(
            collective_id=0,
            vmem_limit_bytes=100 * 1024 * 1024,
        ),
    )(o, Wo)
